# baseline (speedup 1.0000x reference)
import functools

import jax
import jax.numpy as jnp
from jax import lax
from jax.experimental import pallas as pl
from jax.experimental.pallas import tpu as pltpu
from jax.experimental.pallas import tpu_sc as plsc

N_NODES = 10000
N_PAD = 10240
D_IN = 256
D_HID = 256
D_OUT = 128
N_EDGES = 160000
E_PAD = 163840
EC = 128
EDGE_ROWS = E_PAD // EC
ROWS_PER_TILE = EDGE_ROWS // 16
HALF = ROWS_PER_TILE // 2
NC, NS, L = 2, 16, 16
NODE_ROWS_PER_TILE = N_PAD // NS

_VMESH = plsc.VectorSubcoreMesh(core_axis_name="c", subcore_axis_name="s")


def _deg_body(dst_hbm, iota_hbm, out_hbm, dstv, hist2, iota_v, acc2):
    c = lax.axis_index("c")
    s = lax.axis_index("s")
    zrows = 128 // NS

    z16 = jnp.zeros((L,), jnp.float32)

    def _z(r, _):
        def _zq(q, _):
            hist2[r, pl.ds(q * L, L)] = z16
            return 0

        return lax.fori_loop(0, 128 // L, _zq, 0)

    lax.fori_loop(0, 128, _z, 0)

    pltpu.sync_copy(hist2.at[pl.ds(s * zrows, zrows)],
                    acc2.at[pl.ds(s * zrows, zrows)])

    rows = EDGE_ROWS // (NC * NS)
    base = (c * NS + s) * rows
    pltpu.sync_copy(dst_hbm.at[pl.ds(base, rows)], dstv)
    pltpu.sync_copy(iota_hbm, iota_v)

    ones = jnp.ones((L,), jnp.float32)

    def _row(r, _):
        def _vec(q, _):
            v = dstv[r, pl.ds(q * L, L)]
            hi = lax.shift_right_logical(v, 7)
            lo = lax.bitwise_and(v, 127)
            plsc.addupdate_scatter(hist2, (hi, lo), ones)
            return 0

        return lax.fori_loop(0, EC // L, _vec, 0)

    lax.fori_loop(0, rows, _row, 0)
    plsc.subcore_barrier()

    pltpu.sync_copy(hist2, acc2.at[iota_v.at[0]], add=True)
    plsc.subcore_barrier()

    pltpu.sync_copy(acc2.at[pl.ds(s * zrows, zrows)],
                    out_hbm.at[pl.ds(c * 128 + s * zrows, zrows)])


_deg_kernel = functools.partial(
    pl.kernel,
    out_type=jax.ShapeDtypeStruct((NC * 128, 128), jnp.float32),
    mesh=_VMESH,
    scratch_types=[
        pltpu.VMEM((EDGE_ROWS // (NC * NS), EC), jnp.int32),
        pltpu.VMEM((128, 128), jnp.float32),
        pltpu.VMEM((1, 128), jnp.int32),
        pltpu.VMEM_SHARED((128, 128), jnp.float32),
    ],
    compiler_params=pltpu.CompilerParams(needs_layout_passes=False),
)(_deg_body)


def _agg_body(hp_hbm, src2_hbm, dst_hbm, out_hbm, srcv, dstv, rows2, acc,
              gsem, ssema, ssemb, ssemc, ssemd):
    c = lax.axis_index("c")
    s = lax.axis_index("s")

    nbase = c * N_PAD + s * NODE_ROWS_PER_TILE
    pltpu.sync_copy(hp_hbm.at[pl.ds(nbase, NODE_ROWS_PER_TILE)],
                    acc.at[pl.ds(s * NODE_ROWS_PER_TILE, NODE_ROWS_PER_TILE)])
    plsc.subcore_barrier()

    ebase = c * EDGE_ROWS + s * ROWS_PER_TILE
    dbase = s * 4 * ROWS_PER_TILE
    part = ROWS_PER_TILE // 5

    def _part(h, _):
        pltpu.sync_copy(src2_hbm.at[pl.ds(ebase + h * part, part)], srcv)
        pltpu.sync_copy(dst_hbm.at[pl.ds(dbase + h * 4 * part, 4 * part)],
                        dstv)
        pltpu.async_copy(hp_hbm.at[srcv.at[0]], rows2.at[pl.ds(0, EC)],
                         gsem).wait()

        def _step(j, _):
            p = (j % 2) * EC
            pn = EC - p
            jn = lax.rem(j + 1, part)
            d = pltpu.async_copy(hp_hbm.at[srcv.at[jn]],
                                 rows2.at[pl.ds(pn, EC)], gsem)
            q = EC // 4
            da = pltpu.async_copy(rows2.at[pl.ds(p, q)],
                                  acc.at[dstv.at[4 * j]], ssema, add=True)
            db = pltpu.async_copy(rows2.at[pl.ds(p + q, q)],
                                  acc.at[dstv.at[4 * j + 1]], ssemb, add=True)
            dc = pltpu.async_copy(rows2.at[pl.ds(p + 2 * q, q)],
                                  acc.at[dstv.at[4 * j + 2]], ssemc, add=True)
            dd = pltpu.async_copy(rows2.at[pl.ds(p + 3 * q, q)],
                                  acc.at[dstv.at[4 * j + 3]], ssemd, add=True)
            da.wait()
            db.wait()
            dc.wait()
            dd.wait()
            d.wait()
            return 0

        return lax.fori_loop(0, part, _step, 0)

    lax.fori_loop(0, 5, _part, 0)
    plsc.subcore_barrier()

    pltpu.sync_copy(acc.at[pl.ds(s * NODE_ROWS_PER_TILE, NODE_ROWS_PER_TILE)],
                    out_hbm.at[pl.ds(nbase, NODE_ROWS_PER_TILE)])


_agg_kernel = functools.partial(
    pl.kernel,
    out_type=jax.ShapeDtypeStruct((2 * N_PAD, 128), jnp.float32),
    mesh=_VMESH,
    scratch_types=[
        pltpu.VMEM((ROWS_PER_TILE // 5, EC), jnp.int32),
        pltpu.VMEM((4 * ROWS_PER_TILE // 5, EC // 4), jnp.int32),
        pltpu.VMEM((2 * EC, 128), jnp.float32),
        pltpu.VMEM_SHARED((N_PAD, 128), jnp.float32),
        pltpu.SemaphoreType.DMA,
        pltpu.SemaphoreType.DMA,
        pltpu.SemaphoreType.DMA,
        pltpu.SemaphoreType.DMA,
        pltpu.SemaphoreType.DMA,
    ],
    compiler_params=pltpu.CompilerParams(needs_layout_passes=False),
)(_agg_body)


_BN = 512
_NB = N_PAD // _BN


def _mm1_body(x_ref, w_ref, d0_ref, d1_ref, o_ref):
    dinv = lax.rsqrt(d0_ref[...] + d1_ref[...] + 1.0)
    h = jnp.dot(x_ref[...], w_ref[...], preferred_element_type=jnp.float32)
    o_ref[...] = h * dinv


def _mm1(x, w1, d0, d1):
    return pl.pallas_call(
        _mm1_body,
        grid=(2, _NB),
        in_specs=[
            pl.BlockSpec((_BN, D_IN), lambda h, i: (i, 0)),
            pl.BlockSpec((D_IN, 128), lambda h, i: (0, h)),
            pl.BlockSpec((_BN, 1), lambda h, i: (i, 0)),
            pl.BlockSpec((_BN, 1), lambda h, i: (i, 0)),
        ],
        out_specs=pl.BlockSpec((_BN, 128), lambda h, i: (h * _NB + i, 0)),
        out_shape=jax.ShapeDtypeStruct((2 * N_PAD, 128), jnp.float32),
    )(x, w1, d0, d1)


def _mm2_body(za_ref, zb_ref, w_ref, b_ref, d0_ref, d1_ref, o_ref):
    dinv = lax.rsqrt(d0_ref[...] + d1_ref[...] + 1.0)
    z = jnp.concatenate([za_ref[...], zb_ref[...]], axis=1)
    h = jnp.maximum(z * dinv + b_ref[...], 0.0)
    h = jnp.dot(h, w_ref[...], preferred_element_type=jnp.float32)
    o_ref[...] = h * dinv


def _mm2(z, w2, b1, d0, d1):
    return pl.pallas_call(
        _mm2_body,
        grid=(2, _NB),
        in_specs=[
            pl.BlockSpec((_BN, 128), lambda h, i: (i, 0)),
            pl.BlockSpec((_BN, 128), lambda h, i: (_NB + i, 0)),
            pl.BlockSpec((D_HID, 128), lambda h, i: (0, h)),
            pl.BlockSpec((1, D_HID), lambda h, i: (0, 0)),
            pl.BlockSpec((_BN, 1), lambda h, i: (i, 0)),
            pl.BlockSpec((_BN, 1), lambda h, i: (i, 0)),
        ],
        out_specs=pl.BlockSpec((_BN, 128), lambda h, i: (h * _NB + i, 0)),
        out_shape=jax.ShapeDtypeStruct((2 * N_PAD, 128), jnp.float32),
    )(z, z, w2, b1, d0, d1)


def _mm3_body(za_ref, zb_ref, w_ref, b2_ref, bo_ref, d0_ref, d1_ref, o_ref):
    dinv = lax.rsqrt(d0_ref[...] + d1_ref[...] + 1.0)
    z = jnp.concatenate([za_ref[...], zb_ref[...]], axis=1)
    h = jnp.maximum(z * dinv + b2_ref[...], 0.0)
    o_ref[...] = jnp.dot(h, w_ref[...],
                         preferred_element_type=jnp.float32) + bo_ref[...]


def _mm3(z, wout, b2, bout, d0, d1):
    return pl.pallas_call(
        _mm3_body,
        grid=(_NB,),
        in_specs=[
            pl.BlockSpec((_BN, 128), lambda i: (i, 0)),
            pl.BlockSpec((_BN, 128), lambda i: (_NB + i, 0)),
            pl.BlockSpec((D_HID, D_OUT), lambda i: (0, 0)),
            pl.BlockSpec((1, D_HID), lambda i: (0, 0)),
            pl.BlockSpec((1, D_OUT), lambda i: (0, 0)),
            pl.BlockSpec((_BN, 1), lambda i: (i, 0)),
            pl.BlockSpec((_BN, 1), lambda i: (i, 0)),
        ],
        out_specs=pl.BlockSpec((_BN, D_OUT), lambda i: (i, 0)),
        out_shape=jax.ShapeDtypeStruct((N_PAD, D_OUT), jnp.float32),
    )(z, z, wout, b2, bout, d0, d1)


@jax.jit
def kernel(x, edge_index, W1, b1, W2, b2, Wout, bout):
    src = edge_index[0].astype(jnp.int32)
    dst = edge_index[1].astype(jnp.int32)

    pad = jnp.full((E_PAD - N_EDGES,), N_PAD - 1, jnp.int32)
    src_p = jnp.concatenate([src, pad]).reshape(EDGE_ROWS, EC)
    dst_p = jnp.concatenate([dst, pad]).reshape(EDGE_ROWS, EC)
    src2 = jnp.concatenate([src_p, src_p + N_PAD], axis=0)

    x_pad = jnp.pad(x, ((0, N_PAD - N_NODES), (0, 0)))
    iota = jnp.arange(128, dtype=jnp.int32).reshape(1, 128)

    deg_p = _deg_kernel(dst_p, iota)
    d0 = deg_p[:128].reshape(-1)[:N_PAD].reshape(N_PAD, 1)
    d1 = deg_p[128:].reshape(-1)[:N_PAD].reshape(N_PAD, 1)

    b1r = b1.reshape(1, D_HID)
    b2r = b2.reshape(1, D_HID)
    boutr = bout.reshape(1, D_OUT)

    hp1 = _mm1(x_pad, W1, d0, d1)
    dst32 = dst_p.reshape(4 * EDGE_ROWS, EC // 4)
    z1 = _agg_kernel(hp1, src2, dst32)
    hp2 = _mm2(z1, W2, b1r, d0, d1)
    z2 = _agg_kernel(hp2, src2, dst32)
    out = _mm3(z2, Wout, b2r, boutr, d0, d1)
    return out[:N_NODES]


# speedup vs baseline: 1.0395x; 1.0395x over previous
import functools

import jax
import jax.numpy as jnp
from jax import lax
from jax.experimental import pallas as pl
from jax.experimental.pallas import tpu as pltpu
from jax.experimental.pallas import tpu_sc as plsc

N_NODES = 10000
N_PAD = 10240
D_IN = 256
D_HID = 256
D_OUT = 128
N_EDGES = 160000
E_PAD = 163840
EC = 128
EDGE_ROWS = E_PAD // EC
ROWS_PER_TILE = EDGE_ROWS // 16
HALF = ROWS_PER_TILE // 2
NC, NS, L = 2, 16, 16
NODE_ROWS_PER_TILE = N_PAD // NS

_VMESH = plsc.VectorSubcoreMesh(core_axis_name="c", subcore_axis_name="s")


def _deg_body(dst_hbm, iota_hbm, out_hbm, dstv, hist2, iota_v, acc2):
    c = lax.axis_index("c")
    s = lax.axis_index("s")
    zrows = 128 // NS

    z16 = jnp.zeros((L,), jnp.float32)

    def _z(r, _):
        def _zq(q, _):
            hist2[r, pl.ds(q * L, L)] = z16
            return 0

        return lax.fori_loop(0, 128 // L, _zq, 0)

    lax.fori_loop(0, 128, _z, 0)

    pltpu.sync_copy(hist2.at[pl.ds(s * zrows, zrows)],
                    acc2.at[pl.ds(s * zrows, zrows)])

    rows = EDGE_ROWS // (NC * NS)
    base = (c * NS + s) * rows
    pltpu.sync_copy(dst_hbm.at[pl.ds(base, rows)], dstv)
    pltpu.sync_copy(iota_hbm, iota_v)

    ones = jnp.ones((L,), jnp.float32)

    def _row(r, _):
        def _vec(q, _):
            v = dstv[r, pl.ds(q * L, L)]
            hi = lax.shift_right_logical(v, 7)
            lo = lax.bitwise_and(v, 127)
            plsc.addupdate_scatter(hist2, (hi, lo), ones)
            return 0

        return lax.fori_loop(0, EC // L, _vec, 0)

    lax.fori_loop(0, rows, _row, 0)
    plsc.subcore_barrier()

    pltpu.sync_copy(hist2, acc2.at[iota_v.at[0]], add=True)
    plsc.subcore_barrier()

    pltpu.sync_copy(acc2.at[pl.ds(s * zrows, zrows)],
                    out_hbm.at[pl.ds(c * 128 + s * zrows, zrows)])


_deg_kernel = functools.partial(
    pl.kernel,
    out_type=jax.ShapeDtypeStruct((NC * 128, 128), jnp.float32),
    mesh=_VMESH,
    scratch_types=[
        pltpu.VMEM((EDGE_ROWS // (NC * NS), EC), jnp.int32),
        pltpu.VMEM((128, 128), jnp.float32),
        pltpu.VMEM((1, 128), jnp.int32),
        pltpu.VMEM_SHARED((128, 128), jnp.float32),
    ],
    compiler_params=pltpu.CompilerParams(needs_layout_passes=False),
)(_deg_body)


def _agg_body(hp_hbm, src2_hbm, dst_hbm, out_hbm, srcv, dstv, rows2, acc,
              gsem, ssema, ssemb):
    c = lax.axis_index("c")
    s = lax.axis_index("s")

    nbase = c * N_PAD + s * NODE_ROWS_PER_TILE
    pltpu.sync_copy(hp_hbm.at[pl.ds(nbase, NODE_ROWS_PER_TILE)],
                    acc.at[pl.ds(s * NODE_ROWS_PER_TILE, NODE_ROWS_PER_TILE)])
    plsc.subcore_barrier()

    ebase = c * EDGE_ROWS + s * ROWS_PER_TILE
    dbase = s * 2 * ROWS_PER_TILE

    def _half(h, _):
        pltpu.sync_copy(src2_hbm.at[pl.ds(ebase + h * HALF, HALF)], srcv)
        pltpu.sync_copy(dst_hbm.at[pl.ds(dbase + h * 2 * HALF, 2 * HALF)],
                        dstv)
        pltpu.async_copy(hp_hbm.at[srcv.at[0]], rows2.at[pl.ds(0, EC)],
                         gsem).wait()

        def _step(j, _):
            p = (j % 2) * EC
            pn = EC - p
            jn = lax.rem(j + 1, HALF)
            d = pltpu.async_copy(hp_hbm.at[srcv.at[jn]],
                                 rows2.at[pl.ds(pn, EC)], gsem)
            da = pltpu.async_copy(rows2.at[pl.ds(p, EC // 2)],
                                  acc.at[dstv.at[2 * j]], ssema, add=True)
            db = pltpu.async_copy(rows2.at[pl.ds(p + EC // 2, EC // 2)],
                                  acc.at[dstv.at[2 * j + 1]], ssemb, add=True)
            da.wait()
            db.wait()
            d.wait()
            return 0

        return lax.fori_loop(0, HALF, _step, 0)

    lax.fori_loop(0, 2, _half, 0)
    plsc.subcore_barrier()

    pltpu.sync_copy(acc.at[pl.ds(s * NODE_ROWS_PER_TILE, NODE_ROWS_PER_TILE)],
                    out_hbm.at[pl.ds(nbase, NODE_ROWS_PER_TILE)])


_agg_kernel = functools.partial(
    pl.kernel,
    out_type=jax.ShapeDtypeStruct((2 * N_PAD, 128), jnp.float32),
    mesh=_VMESH,
    scratch_types=[
        pltpu.VMEM((HALF, EC), jnp.int32),
        pltpu.VMEM((2 * HALF, EC // 2), jnp.int32),
        pltpu.VMEM((2 * EC, 128), jnp.float32),
        pltpu.VMEM_SHARED((N_PAD, 128), jnp.float32),
        pltpu.SemaphoreType.DMA,
        pltpu.SemaphoreType.DMA,
        pltpu.SemaphoreType.DMA,
    ],
    compiler_params=pltpu.CompilerParams(needs_layout_passes=False),
)(_agg_body)


_BN = 512
_NB = N_PAD // _BN


def _mm1_body(x_ref, w_ref, d0_ref, d1_ref, o_ref):
    dinv = lax.rsqrt(d0_ref[...] + d1_ref[...] + 1.0)
    h = jnp.dot(x_ref[...], w_ref[...], preferred_element_type=jnp.float32)
    o_ref[...] = h * dinv


def _mm1(x, w1, d0, d1):
    return pl.pallas_call(
        _mm1_body,
        grid=(2, _NB),
        in_specs=[
            pl.BlockSpec((_BN, D_IN), lambda h, i: (i, 0)),
            pl.BlockSpec((D_IN, 128), lambda h, i: (0, h)),
            pl.BlockSpec((_BN, 1), lambda h, i: (i, 0)),
            pl.BlockSpec((_BN, 1), lambda h, i: (i, 0)),
        ],
        out_specs=pl.BlockSpec((_BN, 128), lambda h, i: (h * _NB + i, 0)),
        out_shape=jax.ShapeDtypeStruct((2 * N_PAD, 128), jnp.float32),
    )(x, w1, d0, d1)


def _mm2_body(za_ref, zb_ref, w_ref, b_ref, d0_ref, d1_ref, o_ref):
    dinv = lax.rsqrt(d0_ref[...] + d1_ref[...] + 1.0)
    z = jnp.concatenate([za_ref[...], zb_ref[...]], axis=1)
    h = jnp.maximum(z * dinv + b_ref[...], 0.0)
    h = jnp.dot(h, w_ref[...], preferred_element_type=jnp.float32)
    o_ref[...] = h * dinv


def _mm2(z, w2, b1, d0, d1):
    return pl.pallas_call(
        _mm2_body,
        grid=(2, _NB),
        in_specs=[
            pl.BlockSpec((_BN, 128), lambda h, i: (i, 0)),
            pl.BlockSpec((_BN, 128), lambda h, i: (_NB + i, 0)),
            pl.BlockSpec((D_HID, 128), lambda h, i: (0, h)),
            pl.BlockSpec((1, D_HID), lambda h, i: (0, 0)),
            pl.BlockSpec((_BN, 1), lambda h, i: (i, 0)),
            pl.BlockSpec((_BN, 1), lambda h, i: (i, 0)),
        ],
        out_specs=pl.BlockSpec((_BN, 128), lambda h, i: (h * _NB + i, 0)),
        out_shape=jax.ShapeDtypeStruct((2 * N_PAD, 128), jnp.float32),
    )(z, z, w2, b1, d0, d1)


def _mm3_body(za_ref, zb_ref, w_ref, b2_ref, bo_ref, d0_ref, d1_ref, o_ref):
    dinv = lax.rsqrt(d0_ref[...] + d1_ref[...] + 1.0)
    z = jnp.concatenate([za_ref[...], zb_ref[...]], axis=1)
    h = jnp.maximum(z * dinv + b2_ref[...], 0.0)
    o_ref[...] = jnp.dot(h, w_ref[...],
                         preferred_element_type=jnp.float32) + bo_ref[...]


def _mm3(z, wout, b2, bout, d0, d1):
    return pl.pallas_call(
        _mm3_body,
        grid=(_NB,),
        in_specs=[
            pl.BlockSpec((_BN, 128), lambda i: (i, 0)),
            pl.BlockSpec((_BN, 128), lambda i: (_NB + i, 0)),
            pl.BlockSpec((D_HID, D_OUT), lambda i: (0, 0)),
            pl.BlockSpec((1, D_HID), lambda i: (0, 0)),
            pl.BlockSpec((1, D_OUT), lambda i: (0, 0)),
            pl.BlockSpec((_BN, 1), lambda i: (i, 0)),
            pl.BlockSpec((_BN, 1), lambda i: (i, 0)),
        ],
        out_specs=pl.BlockSpec((_BN, D_OUT), lambda i: (i, 0)),
        out_shape=jax.ShapeDtypeStruct((N_PAD, D_OUT), jnp.float32),
    )(z, z, wout, b2, bout, d0, d1)


@jax.jit
def kernel(x, edge_index, W1, b1, W2, b2, Wout, bout):
    src = edge_index[0].astype(jnp.int32)
    dst = edge_index[1].astype(jnp.int32)

    pad = jnp.full((E_PAD - N_EDGES,), N_PAD - 1, jnp.int32)
    src_p = jnp.concatenate([src, pad]).reshape(EDGE_ROWS, EC)
    dst_p = jnp.concatenate([dst, pad]).reshape(EDGE_ROWS, EC)
    src2 = jnp.concatenate([src_p, src_p + N_PAD], axis=0)

    x_pad = jnp.pad(x, ((0, N_PAD - N_NODES), (0, 0)))
    iota = jnp.arange(128, dtype=jnp.int32).reshape(1, 128)

    deg_p = _deg_kernel(dst_p, iota)
    d0 = deg_p[:128].reshape(-1)[:N_PAD].reshape(N_PAD, 1)
    d1 = deg_p[128:].reshape(-1)[:N_PAD].reshape(N_PAD, 1)

    b1r = b1.reshape(1, D_HID)
    b2r = b2.reshape(1, D_HID)
    boutr = bout.reshape(1, D_OUT)

    hp1 = _mm1(x_pad, W1, d0, d1)
    dst64 = dst_p.reshape(2 * EDGE_ROWS, EC // 2)
    z1 = _agg_kernel(hp1, src2, dst64)
    hp2 = _mm2(z1, W2, b1r, d0, d1)
    z2 = _agg_kernel(hp2, src2, dst64)
    out = _mm3(z2, Wout, b2r, boutr, d0, d1)
    return out[:N_NODES]


# speedup vs baseline: 1.1011x; 1.0593x over previous
import functools

import jax
import jax.numpy as jnp
from jax import lax
from jax.experimental import pallas as pl
from jax.experimental.pallas import tpu as pltpu
from jax.experimental.pallas import tpu_sc as plsc

N_NODES = 10000
N_PAD = 10240
D_IN = 256
D_HID = 256
D_OUT = 128
N_EDGES = 160000
E_PAD = 163840
EC = 128
EDGE_ROWS = E_PAD // EC
ROWS_PER_TILE = EDGE_ROWS // 16
HALF = ROWS_PER_TILE // 2
NC, NS, L = 2, 16, 16
NODE_ROWS_PER_TILE = N_PAD // NS

_VMESH = plsc.VectorSubcoreMesh(core_axis_name="c", subcore_axis_name="s")


def _deg_body(dst_hbm, iota_hbm, out_hbm, dstv, hist2, iota_v, acc2):
    c = lax.axis_index("c")
    s = lax.axis_index("s")
    zrows = 128 // NS

    z16 = jnp.zeros((L,), jnp.float32)

    def _z(r, _):
        def _zq(q, _):
            hist2[r, pl.ds(q * L, L)] = z16
            return 0

        return lax.fori_loop(0, 128 // L, _zq, 0)

    lax.fori_loop(0, 128, _z, 0)

    pltpu.sync_copy(hist2.at[pl.ds(s * zrows, zrows)],
                    acc2.at[pl.ds(s * zrows, zrows)])

    rows = EDGE_ROWS // (NC * NS)
    base = (c * NS + s) * rows
    pltpu.sync_copy(dst_hbm.at[pl.ds(base, rows)], dstv)
    pltpu.sync_copy(iota_hbm, iota_v)

    ones = jnp.ones((L,), jnp.float32)

    def _row(r, _):
        def _vec(q, _):
            v = dstv[r, pl.ds(q * L, L)]
            hi = lax.shift_right_logical(v, 7)
            lo = lax.bitwise_and(v, 127)
            plsc.addupdate_scatter(hist2, (hi, lo), ones)
            return 0

        return lax.fori_loop(0, EC // L, _vec, 0)

    lax.fori_loop(0, rows, _row, 0)
    plsc.subcore_barrier()

    pltpu.sync_copy(hist2, acc2.at[iota_v.at[0]], add=True)
    plsc.subcore_barrier()

    pltpu.sync_copy(acc2.at[pl.ds(s * zrows, zrows)],
                    out_hbm.at[pl.ds(c * 128 + s * zrows, zrows)])


_deg_kernel = functools.partial(
    pl.kernel,
    out_type=jax.ShapeDtypeStruct((NC * 128, 128), jnp.float32),
    mesh=_VMESH,
    scratch_types=[
        pltpu.VMEM((EDGE_ROWS // (NC * NS), EC), jnp.int32),
        pltpu.VMEM((128, 128), jnp.float32),
        pltpu.VMEM((1, 128), jnp.int32),
        pltpu.VMEM_SHARED((128, 128), jnp.float32),
    ],
    compiler_params=pltpu.CompilerParams(needs_layout_passes=False),
)(_deg_body)


def _agg_body(hp_hbm, src2_hbm, dst_hbm, out_hbm, srcv, dstv, rows2, acc,
              gsem, ssema, ssemb):
    c = lax.axis_index("c")
    s = lax.axis_index("s")

    nbase = c * N_PAD + s * NODE_ROWS_PER_TILE
    pltpu.sync_copy(hp_hbm.at[pl.ds(nbase, NODE_ROWS_PER_TILE)],
                    acc.at[pl.ds(s * NODE_ROWS_PER_TILE, NODE_ROWS_PER_TILE)])
    plsc.subcore_barrier()

    ebase = c * EDGE_ROWS + s * ROWS_PER_TILE
    dbase = s * 2 * ROWS_PER_TILE

    def _half(h, _):
        pltpu.sync_copy(src2_hbm.at[pl.ds(ebase + h * HALF, HALF)], srcv)
        pltpu.sync_copy(dst_hbm.at[pl.ds(dbase + h * 2 * HALF, 2 * HALF)],
                        dstv)
        pltpu.async_copy(hp_hbm.at[srcv.at[0]], rows2.at[pl.ds(0, EC)],
                         gsem).wait()

        def _step(j, _):
            p = (j % 2) * EC
            pn = EC - p
            jn = lax.rem(j + 1, HALF)
            d = pltpu.async_copy(hp_hbm.at[srcv.at[jn]],
                                 rows2.at[pl.ds(pn, EC)], gsem)
            da = pltpu.async_copy(rows2.at[pl.ds(p, EC // 2)],
                                  acc.at[dstv.at[2 * j]], ssema, add=True)
            db = pltpu.async_copy(rows2.at[pl.ds(p + EC // 2, EC // 2)],
                                  acc.at[dstv.at[2 * j + 1]], ssemb, add=True)
            da.wait()
            db.wait()
            d.wait()
            return 0

        return lax.fori_loop(0, HALF, _step, 0)

    lax.fori_loop(0, 2, _half, 0)
    plsc.subcore_barrier()

    pltpu.sync_copy(acc.at[pl.ds(s * NODE_ROWS_PER_TILE, NODE_ROWS_PER_TILE)],
                    out_hbm.at[pl.ds(nbase, NODE_ROWS_PER_TILE)])


_agg_kernel = functools.partial(
    pl.kernel,
    out_type=jax.ShapeDtypeStruct((2 * N_PAD, 128), jnp.float32),
    mesh=_VMESH,
    scratch_types=[
        pltpu.VMEM((HALF, EC), jnp.int32),
        pltpu.VMEM((2 * HALF, EC // 2), jnp.int32),
        pltpu.VMEM((2 * EC, 128), jnp.float32),
        pltpu.VMEM_SHARED((N_PAD, 128), jnp.float32),
        pltpu.SemaphoreType.DMA,
        pltpu.SemaphoreType.DMA,
        pltpu.SemaphoreType.DMA,
    ],
    compiler_params=pltpu.CompilerParams(needs_layout_passes=False),
)(_agg_body)


_BN = 2048
_NB = N_PAD // _BN


def _mm1_body(x_ref, w_ref, d0_ref, d1_ref, o_ref):
    dinv = lax.rsqrt(d0_ref[...] + d1_ref[...] + 1.0)
    h = jnp.dot(x_ref[...], w_ref[...], preferred_element_type=jnp.float32)
    o_ref[...] = h * dinv


def _mm1(x, w1, d0, d1):
    return pl.pallas_call(
        _mm1_body,
        grid=(2, _NB),
        in_specs=[
            pl.BlockSpec((_BN, D_IN), lambda h, i: (i, 0)),
            pl.BlockSpec((D_IN, 128), lambda h, i: (0, h)),
            pl.BlockSpec((_BN, 1), lambda h, i: (i, 0)),
            pl.BlockSpec((_BN, 1), lambda h, i: (i, 0)),
        ],
        out_specs=pl.BlockSpec((_BN, 128), lambda h, i: (h * _NB + i, 0)),
        out_shape=jax.ShapeDtypeStruct((2 * N_PAD, 128), jnp.float32),
    )(x, w1, d0, d1)


def _mm2_body(za_ref, zb_ref, w_ref, b_ref, d0_ref, d1_ref, o_ref):
    dinv = lax.rsqrt(d0_ref[...] + d1_ref[...] + 1.0)
    z = jnp.concatenate([za_ref[...], zb_ref[...]], axis=1)
    h = jnp.maximum(z * dinv + b_ref[...], 0.0)
    h = jnp.dot(h, w_ref[...], preferred_element_type=jnp.float32)
    o_ref[...] = h * dinv


def _mm2(z, w2, b1, d0, d1):
    return pl.pallas_call(
        _mm2_body,
        grid=(2, _NB),
        in_specs=[
            pl.BlockSpec((_BN, 128), lambda h, i: (i, 0)),
            pl.BlockSpec((_BN, 128), lambda h, i: (_NB + i, 0)),
            pl.BlockSpec((D_HID, 128), lambda h, i: (0, h)),
            pl.BlockSpec((1, D_HID), lambda h, i: (0, 0)),
            pl.BlockSpec((_BN, 1), lambda h, i: (i, 0)),
            pl.BlockSpec((_BN, 1), lambda h, i: (i, 0)),
        ],
        out_specs=pl.BlockSpec((_BN, 128), lambda h, i: (h * _NB + i, 0)),
        out_shape=jax.ShapeDtypeStruct((2 * N_PAD, 128), jnp.float32),
    )(z, z, w2, b1, d0, d1)


def _mm3_body(za_ref, zb_ref, w_ref, b2_ref, bo_ref, d0_ref, d1_ref, o_ref):
    dinv = lax.rsqrt(d0_ref[...] + d1_ref[...] + 1.0)
    z = jnp.concatenate([za_ref[...], zb_ref[...]], axis=1)
    h = jnp.maximum(z * dinv + b2_ref[...], 0.0)
    o_ref[...] = jnp.dot(h, w_ref[...],
                         preferred_element_type=jnp.float32) + bo_ref[...]


def _mm3(z, wout, b2, bout, d0, d1):
    return pl.pallas_call(
        _mm3_body,
        grid=(_NB,),
        in_specs=[
            pl.BlockSpec((_BN, 128), lambda i: (i, 0)),
            pl.BlockSpec((_BN, 128), lambda i: (_NB + i, 0)),
            pl.BlockSpec((D_HID, D_OUT), lambda i: (0, 0)),
            pl.BlockSpec((1, D_HID), lambda i: (0, 0)),
            pl.BlockSpec((1, D_OUT), lambda i: (0, 0)),
            pl.BlockSpec((_BN, 1), lambda i: (i, 0)),
            pl.BlockSpec((_BN, 1), lambda i: (i, 0)),
        ],
        out_specs=pl.BlockSpec((_BN, D_OUT), lambda i: (i, 0)),
        out_shape=jax.ShapeDtypeStruct((N_PAD, D_OUT), jnp.float32),
    )(z, z, wout, b2, bout, d0, d1)


@jax.jit
def kernel(x, edge_index, W1, b1, W2, b2, Wout, bout):
    src = edge_index[0].astype(jnp.int32)
    dst = edge_index[1].astype(jnp.int32)

    pad = jnp.full((E_PAD - N_EDGES,), N_PAD - 1, jnp.int32)
    src_p = jnp.concatenate([src, pad]).reshape(EDGE_ROWS, EC)
    dst_p = jnp.concatenate([dst, pad]).reshape(EDGE_ROWS, EC)
    src2 = jnp.concatenate([src_p, src_p + N_PAD], axis=0)

    x_pad = jnp.pad(x, ((0, N_PAD - N_NODES), (0, 0)))
    iota = jnp.arange(128, dtype=jnp.int32).reshape(1, 128)

    deg_p = _deg_kernel(dst_p, iota)
    d0 = deg_p[:128].reshape(-1)[:N_PAD].reshape(N_PAD, 1)
    d1 = deg_p[128:].reshape(-1)[:N_PAD].reshape(N_PAD, 1)

    b1r = b1.reshape(1, D_HID)
    b2r = b2.reshape(1, D_HID)
    boutr = bout.reshape(1, D_OUT)

    hp1 = _mm1(x_pad, W1, d0, d1)
    dst64 = dst_p.reshape(2 * EDGE_ROWS, EC // 2)
    z1 = _agg_kernel(hp1, src2, dst64)
    hp2 = _mm2(z1, W2, b1r, d0, d1)
    z2 = _agg_kernel(hp2, src2, dst64)
    out = _mm3(z2, Wout, b2r, boutr, d0, d1)
    return out[:N_NODES]


# speedup vs baseline: 1.1014x; 1.0003x over previous
import functools

import jax
import jax.numpy as jnp
from jax import lax
from jax.experimental import pallas as pl
from jax.experimental.pallas import tpu as pltpu
from jax.experimental.pallas import tpu_sc as plsc

N_NODES = 10000
N_PAD = 10240
D_IN = 256
D_HID = 256
D_OUT = 128
N_EDGES = 160000
E_PAD = 163840
EC = 128
EDGE_ROWS = E_PAD // EC
ROWS_PER_TILE = EDGE_ROWS // 16
HALF = ROWS_PER_TILE // 2
NC, NS, L = 2, 16, 16
NODE_ROWS_PER_TILE = N_PAD // NS

_VMESH = plsc.VectorSubcoreMesh(core_axis_name="c", subcore_axis_name="s")


def _deg_body(dst_hbm, iota_hbm, out_hbm, dstv, hist2, iota_v, acc2):
    c = lax.axis_index("c")
    s = lax.axis_index("s")
    zrows = 128 // NS

    z16 = jnp.zeros((L,), jnp.float32)

    def _z(r, _):
        def _zq(q, _):
            hist2[r, pl.ds(q * L, L)] = z16
            return 0

        return lax.fori_loop(0, 128 // L, _zq, 0)

    lax.fori_loop(0, 128, _z, 0)

    pltpu.sync_copy(hist2.at[pl.ds(s * zrows, zrows)],
                    acc2.at[pl.ds(s * zrows, zrows)])

    rows = EDGE_ROWS // (NC * NS)
    base = (c * NS + s) * rows
    pltpu.sync_copy(dst_hbm.at[pl.ds(base, rows)], dstv)
    pltpu.sync_copy(iota_hbm, iota_v)

    ones = jnp.ones((L,), jnp.float32)

    def _row(r, _):
        def _vec(q, _):
            v = dstv[r, pl.ds(q * L, L)]
            hi = lax.shift_right_logical(v, 7)
            lo = lax.bitwise_and(v, 127)
            plsc.addupdate_scatter(hist2, (hi, lo), ones)
            return 0

        return lax.fori_loop(0, EC // L, _vec, 0)

    lax.fori_loop(0, rows, _row, 0)
    plsc.subcore_barrier()

    pltpu.sync_copy(hist2, acc2.at[iota_v.at[0]], add=True)
    plsc.subcore_barrier()

    pltpu.sync_copy(acc2.at[pl.ds(s * zrows, zrows)],
                    out_hbm.at[pl.ds(c * 128 + s * zrows, zrows)])


_deg_kernel = functools.partial(
    pl.kernel,
    out_type=jax.ShapeDtypeStruct((NC * 128, 128), jnp.float32),
    mesh=_VMESH,
    scratch_types=[
        pltpu.VMEM((EDGE_ROWS // (NC * NS), EC), jnp.int32),
        pltpu.VMEM((128, 128), jnp.float32),
        pltpu.VMEM((1, 128), jnp.int32),
        pltpu.VMEM_SHARED((128, 128), jnp.float32),
    ],
    compiler_params=pltpu.CompilerParams(needs_layout_passes=False),
)(_deg_body)


def _agg_body(hp_hbm, src2_hbm, dst_hbm, out_hbm, srcv, dstv, rows2, acc,
              gsem, ssema, ssemb):
    c = lax.axis_index("c")
    s = lax.axis_index("s")

    nbase = c * N_PAD + s * NODE_ROWS_PER_TILE
    pltpu.sync_copy(hp_hbm.at[pl.ds(nbase, NODE_ROWS_PER_TILE)],
                    acc.at[pl.ds(s * NODE_ROWS_PER_TILE, NODE_ROWS_PER_TILE)])
    plsc.subcore_barrier()

    ebase = c * EDGE_ROWS + s * ROWS_PER_TILE
    dbase = s * 2 * ROWS_PER_TILE

    def _half(h, _):
        pltpu.sync_copy(src2_hbm.at[pl.ds(ebase + h * HALF, HALF)], srcv)
        pltpu.sync_copy(dst_hbm.at[pl.ds(dbase + h * 2 * HALF, 2 * HALF)],
                        dstv)
        pltpu.async_copy(hp_hbm.at[srcv.at[0]], rows2.at[pl.ds(0, EC)],
                         gsem).wait()

        def _step(j, _):
            p = (j % 2) * EC
            pn = EC - p
            jn = lax.rem(j + 1, HALF)
            d = pltpu.async_copy(hp_hbm.at[srcv.at[jn]],
                                 rows2.at[pl.ds(pn, EC)], gsem)
            da = pltpu.async_copy(rows2.at[pl.ds(p, EC // 2)],
                                  acc.at[dstv.at[2 * j]], ssema, add=True)
            db = pltpu.async_copy(rows2.at[pl.ds(p + EC // 2, EC // 2)],
                                  acc.at[dstv.at[2 * j + 1]], ssemb, add=True)
            da.wait()
            db.wait()
            d.wait()
            return 0

        return lax.fori_loop(0, HALF, _step, 0)

    lax.fori_loop(0, 2, _half, 0)
    plsc.subcore_barrier()

    pltpu.sync_copy(acc.at[pl.ds(s * NODE_ROWS_PER_TILE, NODE_ROWS_PER_TILE)],
                    out_hbm.at[pl.ds(nbase, NODE_ROWS_PER_TILE)])


_agg_kernel = functools.partial(
    pl.kernel,
    out_type=jax.ShapeDtypeStruct((2 * N_PAD, 128), jnp.float32),
    mesh=_VMESH,
    scratch_types=[
        pltpu.VMEM((HALF, EC), jnp.int32),
        pltpu.VMEM((2 * HALF, EC // 2), jnp.int32),
        pltpu.VMEM((2 * EC, 128), jnp.float32),
        pltpu.VMEM_SHARED((N_PAD, 128), jnp.float32),
        pltpu.SemaphoreType.DMA,
        pltpu.SemaphoreType.DMA,
        pltpu.SemaphoreType.DMA,
    ],
    compiler_params=pltpu.CompilerParams(needs_layout_passes=False),
)(_agg_body)


_BN = 5120
_NB = N_PAD // _BN


def _mm1_body(x_ref, w_ref, d0_ref, d1_ref, o_ref):
    dinv = lax.rsqrt(d0_ref[...] + d1_ref[...] + 1.0)
    h = jnp.dot(x_ref[...], w_ref[...], preferred_element_type=jnp.float32)
    o_ref[...] = h * dinv


def _mm1(x, w1, d0, d1):
    return pl.pallas_call(
        _mm1_body,
        grid=(2, _NB),
        in_specs=[
            pl.BlockSpec((_BN, D_IN), lambda h, i: (i, 0)),
            pl.BlockSpec((D_IN, 128), lambda h, i: (0, h)),
            pl.BlockSpec((_BN, 1), lambda h, i: (i, 0)),
            pl.BlockSpec((_BN, 1), lambda h, i: (i, 0)),
        ],
        out_specs=pl.BlockSpec((_BN, 128), lambda h, i: (h * _NB + i, 0)),
        out_shape=jax.ShapeDtypeStruct((2 * N_PAD, 128), jnp.float32),
    )(x, w1, d0, d1)


def _mm2_body(za_ref, zb_ref, w_ref, b_ref, d0_ref, d1_ref, o_ref):
    dinv = lax.rsqrt(d0_ref[...] + d1_ref[...] + 1.0)
    z = jnp.concatenate([za_ref[...], zb_ref[...]], axis=1)
    h = jnp.maximum(z * dinv + b_ref[...], 0.0)
    h = jnp.dot(h, w_ref[...], preferred_element_type=jnp.float32)
    o_ref[...] = h * dinv


def _mm2(z, w2, b1, d0, d1):
    return pl.pallas_call(
        _mm2_body,
        grid=(2, _NB),
        in_specs=[
            pl.BlockSpec((_BN, 128), lambda h, i: (i, 0)),
            pl.BlockSpec((_BN, 128), lambda h, i: (_NB + i, 0)),
            pl.BlockSpec((D_HID, 128), lambda h, i: (0, h)),
            pl.BlockSpec((1, D_HID), lambda h, i: (0, 0)),
            pl.BlockSpec((_BN, 1), lambda h, i: (i, 0)),
            pl.BlockSpec((_BN, 1), lambda h, i: (i, 0)),
        ],
        out_specs=pl.BlockSpec((_BN, 128), lambda h, i: (h * _NB + i, 0)),
        out_shape=jax.ShapeDtypeStruct((2 * N_PAD, 128), jnp.float32),
    )(z, z, w2, b1, d0, d1)


def _mm3_body(za_ref, zb_ref, w_ref, b2_ref, bo_ref, d0_ref, d1_ref, o_ref):
    dinv = lax.rsqrt(d0_ref[...] + d1_ref[...] + 1.0)
    z = jnp.concatenate([za_ref[...], zb_ref[...]], axis=1)
    h = jnp.maximum(z * dinv + b2_ref[...], 0.0)
    o_ref[...] = jnp.dot(h, w_ref[...],
                         preferred_element_type=jnp.float32) + bo_ref[...]


def _mm3(z, wout, b2, bout, d0, d1):
    return pl.pallas_call(
        _mm3_body,
        grid=(_NB,),
        in_specs=[
            pl.BlockSpec((_BN, 128), lambda i: (i, 0)),
            pl.BlockSpec((_BN, 128), lambda i: (_NB + i, 0)),
            pl.BlockSpec((D_HID, D_OUT), lambda i: (0, 0)),
            pl.BlockSpec((1, D_HID), lambda i: (0, 0)),
            pl.BlockSpec((1, D_OUT), lambda i: (0, 0)),
            pl.BlockSpec((_BN, 1), lambda i: (i, 0)),
            pl.BlockSpec((_BN, 1), lambda i: (i, 0)),
        ],
        out_specs=pl.BlockSpec((_BN, D_OUT), lambda i: (i, 0)),
        out_shape=jax.ShapeDtypeStruct((N_PAD, D_OUT), jnp.float32),
    )(z, z, wout, b2, bout, d0, d1)


@jax.jit
def kernel(x, edge_index, W1, b1, W2, b2, Wout, bout):
    src = edge_index[0].astype(jnp.int32)
    dst = edge_index[1].astype(jnp.int32)

    pad = jnp.full((E_PAD - N_EDGES,), N_PAD - 1, jnp.int32)
    src_p = jnp.concatenate([src, pad]).reshape(EDGE_ROWS, EC)
    dst_p = jnp.concatenate([dst, pad]).reshape(EDGE_ROWS, EC)
    src2 = jnp.concatenate([src_p, src_p + N_PAD], axis=0)

    x_pad = jnp.pad(x, ((0, N_PAD - N_NODES), (0, 0)))
    iota = jnp.arange(128, dtype=jnp.int32).reshape(1, 128)

    deg_p = _deg_kernel(dst_p, iota)
    d0 = deg_p[:128].reshape(-1)[:N_PAD].reshape(N_PAD, 1)
    d1 = deg_p[128:].reshape(-1)[:N_PAD].reshape(N_PAD, 1)

    b1r = b1.reshape(1, D_HID)
    b2r = b2.reshape(1, D_HID)
    boutr = bout.reshape(1, D_OUT)

    hp1 = _mm1(x_pad, W1, d0, d1)
    dst64 = dst_p.reshape(2 * EDGE_ROWS, EC // 2)
    z1 = _agg_kernel(hp1, src2, dst64)
    hp2 = _mm2(z1, W2, b1r, d0, d1)
    z2 = _agg_kernel(hp2, src2, dst64)
    out = _mm3(z2, Wout, b2r, boutr, d0, d1)
    return out[:N_NODES]
